# Initial kernel scaffold; baseline (speedup 1.0000x reference)
import jax
import jax.numpy as jnp
from jax.experimental import pallas as pl


def kernel(x_input, x_mask):
    raise NotImplementedError("write your pallas kernel here")


# speedup vs baseline: 2.6011x; 2.6011x over previous
import functools

import jax
import jax.numpy as jnp
import numpy as np
from jax.experimental import pallas as pl
from jax.experimental.pallas import tpu as pltpu

_ALPHA = 0.9
_MCL = 10
_L = 809
_D = 128
_B = 4
_LP = 832
_TR = 160
_WIN = 32


def _tri(n):
    return n * (n + 1) // 2


def _cand_ij(doc_len, mcl):
    n_cand = (doc_len - mcl) * mcl + _tri(mcl)
    ii = np.zeros(n_cand, np.int64)
    jj = np.zeros(n_cand, np.int64)
    for i in range(doc_len):
        if i < doc_len - mcl:
            s = i * mcl
            e = s + mcl
        else:
            rev = doc_len - i - 1
            base = (doc_len - mcl) * mcl
            s = base + _tri(mcl) - _tri(rev + 1)
            e = base + _tri(mcl) - _tri(rev)
        ii[s:e] = i
        jj[s:e] = np.arange(i, i + (e - s))
    return n_cand, ii, jj


@functools.lru_cache(maxsize=None)
def _constants():
    n_cand, ii, jj = _cand_ij(_L, _MCL)
    t_idx = np.arange(_LP)[:, None]
    l_idx = np.arange(_LP)[None, :]
    valid = (l_idx < _L) & (t_idx <= _L)
    tf = np.where((l_idx <= t_idx - 1) & valid,
                  _ALPHA ** np.maximum(t_idx - 1 - l_idx, 0), 0.0)
    tb = np.where((l_idx >= t_idx) & valid & (t_idx < _L),
                  _ALPHA ** np.maximum(l_idx - t_idx, 0), 0.0)
    n_tiles = (n_cand + _TR - 1) // _TR
    sel_a = np.zeros((n_tiles, _TR, _WIN), np.float32)
    sel_b = np.zeros((n_tiles, _TR, _WIN), np.float32)
    sel_m = np.zeros((n_tiles, _TR, _WIN), np.float32)
    for t in range(n_tiles):
        w0 = 16 * t
        for r in range(_TR):
            c = t * _TR + r
            if c >= n_cand:
                continue
            i, j = ii[c], jj[c]
            oa = j + 1 - w0
            ob = i - w0
            assert 0 <= oa < _WIN and 0 <= ob < _WIN, (t, r, i, j)
            sel_a[t, r, oa] = 1.0
            sel_b[t, r, ob] = 1.0
            sel_m[t, r, oa] += 1.0
            sel_m[t, r, ob] -= _ALPHA ** (j - i + 1)
    return (n_cand, n_tiles,
            jnp.asarray(tf, jnp.float32), jnp.asarray(tb, jnp.float32),
            jnp.asarray(sel_a), jnp.asarray(sel_b), jnp.asarray(sel_m))


def _scan_body(xp_ref, tf_ref, tb_ref, fp_ref, bk_ref):
    x = xp_ref[0]
    fp_ref[0] = jax.lax.dot(tf_ref[...], x,
                            preferred_element_type=jnp.float32)
    bk_ref[0] = jax.lax.dot(tb_ref[...], x,
                            preferred_element_type=jnp.float32)


def _expand_body(fp_ref, bk_ref, sa_ref, sb_ref, sm_ref, out_ref):
    t = pl.program_id(0)
    w0 = t * 16
    sa = sa_ref[0]
    sb = sb_ref[0]
    sm = sm_ref[0]
    for b in range(_B):
        fw = fp_ref[b, pl.ds(w0, _WIN), :]
        bw = bk_ref[b, pl.ds(w0, _WIN), :]
        dot = functools.partial(jax.lax.dot,
                                preferred_element_type=jnp.float32)
        out_ref[b, :, 0:_D] = dot(sm, fw)
        out_ref[b, :, _D:2 * _D] = dot(sb, fw)
        out_ref[b, :, 2 * _D:3 * _D] = dot(sa, fw)
        out_ref[b, :, 3 * _D:4 * _D] = dot(sa, bw)
        out_ref[b, :, 4 * _D:5 * _D] = dot(sb, bw)


def kernel(x_input, x_mask):
    del x_mask
    n_cand, n_tiles, tf, tb, sel_a, sel_b, sel_m = _constants()
    xp = jnp.pad(x_input, ((0, 0), (0, _LP - _L), (0, 0)))

    fp, bk = pl.pallas_call(
        _scan_body,
        grid=(_B,),
        in_specs=[
            pl.BlockSpec((1, _LP, _D), lambda b: (b, 0, 0)),
            pl.BlockSpec((_LP, _LP), lambda b: (0, 0)),
            pl.BlockSpec((_LP, _LP), lambda b: (0, 0)),
        ],
        out_specs=[
            pl.BlockSpec((1, _LP, _D), lambda b: (b, 0, 0)),
            pl.BlockSpec((1, _LP, _D), lambda b: (b, 0, 0)),
        ],
        out_shape=[
            jax.ShapeDtypeStruct((_B, _LP, _D), jnp.float32),
            jax.ShapeDtypeStruct((_B, _LP, _D), jnp.float32),
        ],
    )(xp, tf, tb)

    out = pl.pallas_call(
        _expand_body,
        grid=(n_tiles,),
        in_specs=[
            pl.BlockSpec((_B, _LP, _D), lambda t: (0, 0, 0)),
            pl.BlockSpec((_B, _LP, _D), lambda t: (0, 0, 0)),
            pl.BlockSpec((1, _TR, _WIN), lambda t: (t, 0, 0)),
            pl.BlockSpec((1, _TR, _WIN), lambda t: (t, 0, 0)),
            pl.BlockSpec((1, _TR, _WIN), lambda t: (t, 0, 0)),
        ],
        out_specs=pl.BlockSpec((_B, _TR, 5 * _D), lambda t: (0, t, 0)),
        out_shape=jax.ShapeDtypeStruct((_B, n_cand, 5 * _D), jnp.float32),
    )(fp, bk, sel_a, sel_b, sel_m)
    return out
